# Initial kernel scaffold; baseline (speedup 1.0000x reference)
import functools

import jax
import jax.numpy as jnp
from jax import lax
from jax.experimental import pallas as pl
from jax.experimental.pallas import tpu as pltpu
from jax.experimental.pallas import tpu_sc as plsc

N = 100000
E = 6400000
K = 8

NS = 16
NC = 2
NT = N // NS
ER = E // 128
ERT = ER // NS
KCH = 25
NCH = ERT // KCH


def _rsqrt_newton(d):
    i = plsc.bitcast(d, jnp.int32)
    i = 0x5F3759DF - lax.shift_right_logical(i, 1)
    y = plsc.bitcast(i, jnp.float32)
    for _ in range(3):
        y = y * (1.5 - 0.5 * d * y * y)
    return y


def _sc_body(x_hbm, e3_hbm, z8_hbm, z1_hbm, one_hbm,
             agg_hbm, dis_hbm,
             deg_s, z_s, acc_s,
             gi_v, si_v, msg_v, one_v, nx_v, nd_v):
    c = lax.axis_index("c")
    s = lax.axis_index("s")
    n0 = s * NT
    e0 = s * ERT

    pltpu.sync_copy(one_hbm, one_v)
    pltpu.sync_copy(z8_hbm, nx_v)
    pltpu.sync_copy(nx_v, acc_s.at[pl.ds(n0, NT), :])
    pltpu.sync_copy(z1_hbm, nd_v.at[pl.ds(0, NT)])
    pltpu.sync_copy(nd_v.at[pl.ds(0, NT)], deg_s.at[pl.ds(n0, NT)])
    plsc.subcore_barrier()

    def deg_chunk(i, _):
        base = e0 + i * KCH
        pltpu.sync_copy(e3_hbm.at[0, pl.ds(base, KCH), :], gi_v)

        def inner(j, _):
            pltpu.sync_copy(one_v, deg_s.at[gi_v.at[j]], add=True)
            return 0

        return lax.fori_loop(0, KCH, inner, 0)

    lax.fori_loop(0, NCH, deg_chunk, 0)
    plsc.subcore_barrier()

    pltpu.sync_copy(deg_s.at[pl.ds(n0, NT)], nd_v.at[pl.ds(0, NT)])

    def dis_step(i, _):
        d = nd_v[pl.ds(i * 16, 16)]
        nd_v[pl.ds(i * 16, 16)] = _rsqrt_newton(d)
        return 0

    lax.fori_loop(0, (NT + 15) // 16, dis_step, 0)

    pltpu.sync_copy(x_hbm.at[pl.ds(n0, NT), :], nx_v)
    nxf = nx_v.reshape(NT * K)
    iota16 = lax.broadcasted_iota(jnp.int32, (16,), 0)

    def z_step(j, _):
        xv = nxf[pl.ds(j * 16, 16)]
        dv = plsc.load_gather(nd_v, [2 * j + iota16 // 8])
        nxf[pl.ds(j * 16, 16)] = jnp.maximum(xv, 0.0) * dv
        return 0

    lax.fori_loop(0, NT * K // 16, z_step, 0)

    pltpu.sync_copy(nx_v, z_s.at[pl.ds(n0, NT), :])

    @pl.when(c == 0)
    def _():
        pltpu.sync_copy(nd_v.at[pl.ds(0, NT)], dis_hbm.at[pl.ds(n0, NT)])

    plsc.subcore_barrier()

    gdim = c
    sdim = 1 - c

    def agg_chunk(i, _):
        base = e0 + i * KCH
        pltpu.sync_copy(e3_hbm.at[gdim, pl.ds(base, KCH), :], gi_v)
        pltpu.sync_copy(e3_hbm.at[sdim, pl.ds(base, KCH), :], si_v)

        def inner(j, _):
            pltpu.sync_copy(z_s.at[gi_v.at[j]], msg_v)
            pltpu.sync_copy(msg_v, acc_s.at[si_v.at[j]], add=True)
            return 0

        return lax.fori_loop(0, KCH, inner, 0)

    lax.fori_loop(0, NCH, agg_chunk, 0)
    plsc.subcore_barrier()

    pltpu.sync_copy(acc_s.at[pl.ds(n0, NT), :], nx_v)
    pltpu.sync_copy(nx_v, agg_hbm.at[c, pl.ds(n0, NT), :])


_sc_call = pl.kernel(
    _sc_body,
    out_type=(
        jax.ShapeDtypeStruct((NC, N, K), jnp.float32),
        jax.ShapeDtypeStruct((N,), jnp.float32),
    ),
    mesh=plsc.VectorSubcoreMesh(core_axis_name="c", subcore_axis_name="s"),
    scratch_types=(
        pltpu.VMEM_SHARED((N,), jnp.float32),
        pltpu.VMEM_SHARED((N, K), jnp.float32),
        pltpu.VMEM_SHARED((N, K), jnp.float32),
        pltpu.VMEM((KCH, 128), jnp.int32),
        pltpu.VMEM((KCH, 128), jnp.int32),
        pltpu.VMEM((128, K), jnp.float32),
        pltpu.VMEM((128,), jnp.float32),
        pltpu.VMEM((NT, K), jnp.float32),
        pltpu.VMEM((NT + 16,), jnp.float32),
    ),
)


def _tc_body(ai_ref, ao_ref, dis_ref, wit_ref, wot_ref, oi_ref, oo_ref):
    dis = dis_ref[...]
    oi_ref[...] = dis * jnp.dot(ai_ref[...], wit_ref[...],
                                preferred_element_type=jnp.float32)
    oo_ref[...] = dis * jnp.dot(ao_ref[...], wot_ref[...],
                                preferred_element_type=jnp.float32)


BN = 5000


def _tc_call(ai, ao, dis, wit, wot):
    grid = (N // BN,)
    return pl.pallas_call(
        _tc_body,
        grid=grid,
        in_specs=[
            pl.BlockSpec((BN, K), lambda i: (i, 0)),
            pl.BlockSpec((BN, K), lambda i: (i, 0)),
            pl.BlockSpec((BN, 1), lambda i: (i, 0)),
            pl.BlockSpec((K, K), lambda i: (0, 0)),
            pl.BlockSpec((K, K), lambda i: (0, 0)),
        ],
        out_specs=[
            pl.BlockSpec((BN, K), lambda i: (i, 0)),
            pl.BlockSpec((BN, K), lambda i: (i, 0)),
        ],
        out_shape=[
            jax.ShapeDtypeStruct((N, K), jnp.float32),
            jax.ShapeDtypeStruct((N, K), jnp.float32),
        ],
    )(ai, ao, dis, wit, wot)


def kernel(x, edge_index, W_in, W_out):
    e3 = edge_index.astype(jnp.int32).reshape(2, ER, 128)
    z8 = jnp.zeros((NT, K), jnp.float32)
    z1 = jnp.zeros((NT,), jnp.float32)
    one = jnp.ones((128,), jnp.float32)
    agg, dis = _sc_call(x, e3, z8, z1, one)
    oi, oo = _tc_call(agg[0], agg[1], dis.reshape(N, 1),
                      W_in.T, W_out.T)
    return jnp.concatenate([oi, oo], axis=1)


# speedup vs baseline: 28.5660x; 28.5660x over previous
import jax
import jax.numpy as jnp
from jax import lax
from jax.experimental import pallas as pl
from jax.experimental.pallas import tpu as pltpu
from jax.experimental.pallas import tpu_sc as plsc

N = 100000
E = 6400000
K = 8

NS = 16
NC = 2
NP = 100096
NT = NP // NS
ER = E // 128
ERT = ER // NS
KCH = 25
NCH = ERT // KCH


def _rsqrt_newton(d):
    i = plsc.bitcast(d, jnp.int32)
    i = 0x5F3759DF - lax.shift_right_logical(i, 1)
    y = plsc.bitcast(i, jnp.float32)
    for _ in range(3):
        y = y * (1.5 - 0.5 * d * y * y)
    return y


def _sc_body(*refs):
    (x0, x1, x2, x3, x4, x5, x6, x7, e4_hbm, zro_hbm, one_hbm) = refs[:11]
    outs = refs[11:11 + 2 * K]
    dis_hbm = refs[11 + 2 * K]
    deg_s = refs[12 + 2 * K]
    zs = refs[13 + 2 * K:13 + 3 * K]
    accs = refs[13 + 3 * K:13 + 4 * K]
    gi_v, si_v, pay_v, one_v, xb_v, nd_v = refs[13 + 4 * K:]
    xks = (x0, x1, x2, x3, x4, x5, x6, x7)

    c = lax.axis_index("c")
    s = lax.axis_index("s")
    n0 = s * NT
    e0 = s * ERT

    pltpu.sync_copy(one_hbm, one_v)
    pltpu.sync_copy(zro_hbm, xb_v)
    pltpu.sync_copy(xb_v, deg_s.at[pl.ds(n0, NT)])
    for k in range(K):
        pltpu.sync_copy(xb_v, accs[k].at[pl.ds(n0, NT)])
    plsc.subcore_barrier()

    def deg_chunk(i, _):
        base = e0 + i * KCH
        pltpu.sync_copy(e4_hbm.at[0, pl.ds(base, KCH), :, :], gi_v)

        def inner(j, _):
            pltpu.sync_copy(one_v, deg_s.at[gi_v.at[j, 0]], add=True)
            return 0

        return lax.fori_loop(0, KCH, inner, 0)

    lax.fori_loop(0, NCH, deg_chunk, 0)
    plsc.subcore_barrier()

    pltpu.sync_copy(deg_s.at[pl.ds(n0, NT)], nd_v)

    def dis_step(i, _):
        d = nd_v[pl.ds(i * 16, 16)]
        nd_v[pl.ds(i * 16, 16)] = _rsqrt_newton(d)
        return 0

    lax.fori_loop(0, NT // 16, dis_step, 0)

    @pl.when(c == 0)
    def _():
        pltpu.sync_copy(nd_v, dis_hbm.at[pl.ds(n0, NT)])

    for k in range(K):
        pltpu.sync_copy(xks[k].at[pl.ds(n0, NT)], xb_v)

        def z_step(i, _):
            xv = xb_v[pl.ds(i * 16, 16)]
            dv = nd_v[pl.ds(i * 16, 16)]
            xb_v[pl.ds(i * 16, 16)] = jnp.maximum(xv, 0.0) * dv
            return 0

        lax.fori_loop(0, NT // 16, z_step, 0)
        pltpu.sync_copy(xb_v, zs[k].at[pl.ds(n0, NT)])
    plsc.subcore_barrier()

    gdim = c
    sdim = 1 - c

    def agg_chunk(i, _):
        base = e0 + i * KCH
        pltpu.sync_copy(e4_hbm.at[gdim, pl.ds(base, KCH), :, :], gi_v)
        pltpu.sync_copy(e4_hbm.at[sdim, pl.ds(base, KCH), :, :], si_v)

        def inner(j, _):
            for k in range(K):
                pltpu.sync_copy(zs[k].at[gi_v.at[j, 0]], pay_v)
                pltpu.sync_copy(pay_v, accs[k].at[si_v.at[j, 0]], add=True)
            return 0

        return lax.fori_loop(0, KCH, inner, 0)

    lax.fori_loop(0, NCH, agg_chunk, 0)
    plsc.subcore_barrier()

    @pl.when(c == 0)
    def _():
        for k in range(K):
            pltpu.sync_copy(accs[k].at[pl.ds(n0, NT)], xb_v)
            pltpu.sync_copy(xb_v, outs[k].at[pl.ds(n0, NT)])

    @pl.when(c == 1)
    def _():
        for k in range(K):
            pltpu.sync_copy(accs[k].at[pl.ds(n0, NT)], xb_v)
            pltpu.sync_copy(xb_v, outs[K + k].at[pl.ds(n0, NT)])


_sc_call = pl.kernel(
    _sc_body,
    out_type=tuple(jax.ShapeDtypeStruct((NP,), jnp.float32)
                   for _ in range(2 * K + 1)),
    mesh=plsc.VectorSubcoreMesh(core_axis_name="c", subcore_axis_name="s"),
    compiler_params=pltpu.CompilerParams(needs_layout_passes=False),
    scratch_types=(
        (pltpu.VMEM_SHARED((NP,), jnp.float32),)
        + tuple(pltpu.VMEM_SHARED((NP,), jnp.float32) for _ in range(K))
        + tuple(pltpu.VMEM_SHARED((NP,), jnp.float32) for _ in range(K))
        + (
            pltpu.VMEM((KCH, 1, 128), jnp.int32),
            pltpu.VMEM((KCH, 1, 128), jnp.int32),
            pltpu.VMEM((128,), jnp.float32),
            pltpu.VMEM((128,), jnp.float32),
            pltpu.VMEM((NT,), jnp.float32),
            pltpu.VMEM((NT,), jnp.float32),
        )
    ),
)


def _tc_body(ai_ref, ao_ref, dis_ref, wit_ref, wot_ref, oi_ref, oo_ref):
    dis = dis_ref[...]
    oi_ref[...] = dis * jnp.dot(ai_ref[...], wit_ref[...],
                                preferred_element_type=jnp.float32)
    oo_ref[...] = dis * jnp.dot(ao_ref[...], wot_ref[...],
                                preferred_element_type=jnp.float32)


BN = 5000


def _tc_call(ai, ao, dis, wit, wot):
    grid = (N // BN,)
    return pl.pallas_call(
        _tc_body,
        grid=grid,
        in_specs=[
            pl.BlockSpec((BN, K), lambda i: (i, 0)),
            pl.BlockSpec((BN, K), lambda i: (i, 0)),
            pl.BlockSpec((BN, 1), lambda i: (i, 0)),
            pl.BlockSpec((K, K), lambda i: (0, 0)),
            pl.BlockSpec((K, K), lambda i: (0, 0)),
        ],
        out_specs=[
            pl.BlockSpec((BN, K), lambda i: (i, 0)),
            pl.BlockSpec((BN, K), lambda i: (i, 0)),
        ],
        out_shape=[
            jax.ShapeDtypeStruct((N, K), jnp.float32),
            jax.ShapeDtypeStruct((N, K), jnp.float32),
        ],
    )(ai, ao, dis, wit, wot)


def kernel(x, edge_index, W_in, W_out):
    e4 = edge_index.astype(jnp.int32).reshape(2, ER, 1, 128)
    xp = jnp.pad(x, ((0, NP - N), (0, 0)))
    xks = [xp[:, k] for k in range(K)]
    zro = jnp.zeros((NT,), jnp.float32)
    one = jnp.ones((128,), jnp.float32)
    res = _sc_call(*xks, e4, zro, one)
    agg_in = jnp.stack(res[:K], axis=1)[:N]
    agg_out = jnp.stack(res[K:2 * K], axis=1)[:N]
    dis = res[2 * K][:N].reshape(N, 1)
    oi, oo = _tc_call(agg_in, agg_out, dis, W_in.T, W_out.T)
    return jnp.concatenate([oi, oo], axis=1)


# speedup vs baseline: 64.7544x; 2.2668x over previous
import jax
import jax.numpy as jnp
from jax import lax
from jax.experimental import pallas as pl
from jax.experimental.pallas import tpu as pltpu
from jax.experimental.pallas import tpu_sc as plsc

N = 100000
E = 6400000
K = 8

NS = 16
NC = 2
NP = 100096
NT = NP // NS
ER = E // 128
ERT = ER // NS
KCH = 25
NCH = ERT // KCH


def _rsqrt_newton(d):
    i = plsc.bitcast(d, jnp.int32)
    i = 0x5F3759DF - lax.shift_right_logical(i, 1)
    y = plsc.bitcast(i, jnp.float32)
    for _ in range(3):
        y = y * (1.5 - 0.5 * d * y * y)
    return y


def _sc_body(*refs):
    (x0, x1, x2, x3, x4, x5, x6, x7, e4_hbm, zro_hbm, one_hbm) = refs[:11]
    outs = refs[11:11 + 2 * K]
    dis_hbm = refs[11 + 2 * K]
    deg_s = refs[12 + 2 * K]
    zs = refs[13 + 2 * K:13 + 3 * K]
    accs = refs[13 + 3 * K:13 + 4 * K]
    (gi_v, si_v, pay2_v, one_v, xb_v, nd_v,
     dsem, gsem, ssem) = refs[13 + 4 * K:]
    xks = (x0, x1, x2, x3, x4, x5, x6, x7)

    c = lax.axis_index("c")
    s = lax.axis_index("s")
    n0 = s * NT
    e0 = s * ERT

    pltpu.sync_copy(one_hbm, one_v)
    pltpu.sync_copy(zro_hbm, xb_v)
    pltpu.sync_copy(xb_v, deg_s.at[pl.ds(n0, NT)])
    for k in range(K):
        pltpu.sync_copy(xb_v, accs[k].at[pl.ds(n0, NT)])
    plsc.subcore_barrier()

    DEGQ = 16

    def deg_chunk(i, _):
        base = e0 + i * KCH
        pltpu.sync_copy(e4_hbm.at[0, pl.ds(base, KCH), :, :], gi_v)

        def inner(j, _):
            pltpu.async_copy(one_v, deg_s.at[gi_v.at[j, 0]], dsem, add=True)
            jj = i * KCH + j

            @pl.when(jj >= DEGQ)
            def _():
                pltpu.make_async_copy(zro_hbm.at[pl.ds(0, 128)], one_v,
                                      dsem).wait()

            return 0

        return lax.fori_loop(0, KCH, inner, 0)

    lax.fori_loop(0, NCH, deg_chunk, 0)
    for _q in range(DEGQ):
        pltpu.make_async_copy(zro_hbm.at[pl.ds(0, 128)], one_v, dsem).wait()
    plsc.subcore_barrier()

    pltpu.sync_copy(deg_s.at[pl.ds(n0, NT)], nd_v)

    def dis_step(i, _):
        d = nd_v[pl.ds(i * 16, 16)]
        nd_v[pl.ds(i * 16, 16)] = _rsqrt_newton(d)
        return 0

    lax.fori_loop(0, NT // 16, dis_step, 0)

    @pl.when(c == 0)
    def _():
        pltpu.sync_copy(nd_v, dis_hbm.at[pl.ds(n0, NT)])

    for k in range(K):
        pltpu.sync_copy(xks[k].at[pl.ds(n0, NT)], xb_v)

        def z_step(i, _):
            xv = xb_v[pl.ds(i * 16, 16)]
            dv = nd_v[pl.ds(i * 16, 16)]
            xb_v[pl.ds(i * 16, 16)] = jnp.maximum(xv, 0.0) * dv
            return 0

        lax.fori_loop(0, NT // 16, z_step, 0)
        pltpu.sync_copy(xb_v, zs[k].at[pl.ds(n0, NT)])
    plsc.subcore_barrier()

    gdim = c
    sdim = 1 - c

    def agg_chunk(i, _):
        base = e0 + i * KCH
        pltpu.sync_copy(e4_hbm.at[gdim, pl.ds(base, KCH), :, :], gi_v)
        pltpu.sync_copy(e4_hbm.at[sdim, pl.ds(base, KCH), :, :], si_v)

        def inner(j, _):
            jj = i * KCH + j
            b = lax.rem(jj, 2)

            @pl.when(jj >= 2)
            def _():
                for _q in range(K):
                    pltpu.make_async_copy(zro_hbm.at[pl.ds(0, 128)],
                                          pay2_v.at[0, 0], ssem).wait()

            descs = [
                pltpu.async_copy(zs[k].at[gi_v.at[j, 0]],
                                 pay2_v.at[b * K + k, 0], gsem)
                for k in range(K)
            ]
            for d in descs:
                d.wait()
            for k in range(K):
                pltpu.async_copy(pay2_v.at[b * K + k, 0],
                                 accs[k].at[si_v.at[j, 0]], ssem, add=True)
            return 0

        return lax.fori_loop(0, KCH, inner, 0)

    lax.fori_loop(0, NCH, agg_chunk, 0)
    for _q in range(2 * K):
        pltpu.make_async_copy(zro_hbm.at[pl.ds(0, 128)], pay2_v.at[0, 0],
                              ssem).wait()
    plsc.subcore_barrier()

    @pl.when(c == 0)
    def _():
        for k in range(K):
            pltpu.sync_copy(accs[k].at[pl.ds(n0, NT)], xb_v)
            pltpu.sync_copy(xb_v, outs[k].at[pl.ds(n0, NT)])

    @pl.when(c == 1)
    def _():
        for k in range(K):
            pltpu.sync_copy(accs[k].at[pl.ds(n0, NT)], xb_v)
            pltpu.sync_copy(xb_v, outs[K + k].at[pl.ds(n0, NT)])


_sc_call = pl.kernel(
    _sc_body,
    out_type=tuple(jax.ShapeDtypeStruct((NP,), jnp.float32)
                   for _ in range(2 * K + 1)),
    mesh=plsc.VectorSubcoreMesh(core_axis_name="c", subcore_axis_name="s"),
    compiler_params=pltpu.CompilerParams(needs_layout_passes=False),
    scratch_types=(
        (pltpu.VMEM_SHARED((NP,), jnp.float32),)
        + tuple(pltpu.VMEM_SHARED((NP,), jnp.float32) for _ in range(K))
        + tuple(pltpu.VMEM_SHARED((NP,), jnp.float32) for _ in range(K))
        + (
            pltpu.VMEM((KCH, 1, 128), jnp.int32),
            pltpu.VMEM((KCH, 1, 128), jnp.int32),
            pltpu.VMEM((2 * K, 1, 128), jnp.float32),
            pltpu.VMEM((128,), jnp.float32),
            pltpu.VMEM((NT,), jnp.float32),
            pltpu.VMEM((NT,), jnp.float32),
            pltpu.SemaphoreType.DMA,
            pltpu.SemaphoreType.DMA,
            pltpu.SemaphoreType.DMA,
        )
    ),
)


def _tc_body(ai_ref, ao_ref, dis_ref, wit_ref, wot_ref, oi_ref, oo_ref):
    dis = dis_ref[...]
    oi_ref[...] = dis * jnp.dot(ai_ref[...], wit_ref[...],
                                preferred_element_type=jnp.float32)
    oo_ref[...] = dis * jnp.dot(ao_ref[...], wot_ref[...],
                                preferred_element_type=jnp.float32)


BN = 5000


def _tc_call(ai, ao, dis, wit, wot):
    grid = (N // BN,)
    return pl.pallas_call(
        _tc_body,
        grid=grid,
        in_specs=[
            pl.BlockSpec((BN, K), lambda i: (i, 0)),
            pl.BlockSpec((BN, K), lambda i: (i, 0)),
            pl.BlockSpec((BN, 1), lambda i: (i, 0)),
            pl.BlockSpec((K, K), lambda i: (0, 0)),
            pl.BlockSpec((K, K), lambda i: (0, 0)),
        ],
        out_specs=[
            pl.BlockSpec((BN, K), lambda i: (i, 0)),
            pl.BlockSpec((BN, K), lambda i: (i, 0)),
        ],
        out_shape=[
            jax.ShapeDtypeStruct((N, K), jnp.float32),
            jax.ShapeDtypeStruct((N, K), jnp.float32),
        ],
    )(ai, ao, dis, wit, wot)


def kernel(x, edge_index, W_in, W_out):
    e4 = edge_index.astype(jnp.int32).reshape(2, ER, 1, 128)
    xp = jnp.pad(x, ((0, NP - N), (0, 0)))
    xks = [xp[:, k] for k in range(K)]
    zro = jnp.zeros((NT,), jnp.float32)
    one = jnp.ones((128,), jnp.float32)
    res = _sc_call(*xks, e4, zro, one)
    agg_in = jnp.stack(res[:K], axis=1)[:N]
    agg_out = jnp.stack(res[K:2 * K], axis=1)[:N]
    dis = res[2 * K][:N].reshape(N, 1)
    oi, oo = _tc_call(agg_in, agg_out, dis, W_in.T, W_out.T)
    return jnp.concatenate([oi, oo], axis=1)
